# Initial kernel scaffold; baseline (speedup 1.0000x reference)
import jax
import jax.numpy as jnp
from jax.experimental import pallas as pl


def kernel(x, edge_index, W1, b1, W2, b2, Wih0, Whh0, bih0, bhh0, Wih1, Whh1, bih1, bhh1, fcW, fcb):
    raise NotImplementedError("write your pallas kernel here")


# speedup vs baseline: 14.2779x; 14.2779x over previous
import functools

import jax
import jax.numpy as jnp
from jax import lax
from jax.experimental import pallas as pl
from jax.experimental.pallas import tpu as pltpu
from jax.experimental.pallas import tpu_sc as plsc

NC = 2
NS = 16
NW = NC * NS
CHUNK = 128

N_PAD = 10240

_f32 = jnp.float32


def _sc_degree(dst3, ones_rows, zeros_tbl):
  ch = dst3.shape[1]
  rows_per_tile = N_PAD // NS

  @functools.partial(
      pl.kernel,
      out_type=jax.ShapeDtypeStruct((NC, N_PAD, 16), _f32),
      mesh=plsc.VectorSubcoreMesh(core_axis_name="c", subcore_axis_name="s"),
      scratch_types=[
          pltpu.VMEM((ch, CHUNK), jnp.int32),
          pltpu.VMEM((CHUNK, 16), _f32),
          pltpu.VMEM_SHARED((N_PAD, 16), _f32),
      ],
      compiler_params=pltpu.CompilerParams(use_tc_tiling_on_sc=False),
  )
  def deg_kernel(dst_hbm, ones_hbm, zeros_hbm, out_hbm, idx_v, ones_v, acc):
    c = lax.axis_index("c")
    s = lax.axis_index("s")
    w = s * NC + c
    pltpu.sync_copy(dst_hbm.at[w], idx_v)
    pltpu.sync_copy(ones_hbm, ones_v)
    sl = pl.ds(s * rows_per_tile, rows_per_tile)
    pltpu.sync_copy(zeros_hbm.at[sl], acc.at[sl])
    plsc.subcore_barrier()

    def step(j, carry):
      pltpu.sync_copy(ones_v, acc.at[idx_v.at[j]], add=True)
      return carry

    lax.fori_loop(0, ch, step, 0)
    plsc.subcore_barrier()
    pltpu.sync_copy(acc.at[sl], out_hbm.at[c, sl])

  return deg_kernel(dst3, ones_rows, zeros_tbl)


def _sc_aggregate(y_tbl, src3g, dst3, zeros_tbl):
  d = y_tbl.shape[1]
  ch = dst3.shape[1]
  rows_per_tile = N_PAD // NS

  @functools.partial(
      pl.kernel,
      out_type=jax.ShapeDtypeStruct((NC, N_PAD, d), _f32),
      mesh=plsc.VectorSubcoreMesh(core_axis_name="c", subcore_axis_name="s"),
      scratch_types=[
          pltpu.VMEM((ch + 2, CHUNK), jnp.int32),
          pltpu.VMEM((ch, CHUNK), jnp.int32),
          pltpu.VMEM((2, CHUNK, d), _f32),
          pltpu.VMEM_SHARED((N_PAD, d), _f32),
          pltpu.SemaphoreType.DMA,
          pltpu.SemaphoreType.DMA,
      ],
      compiler_params=pltpu.CompilerParams(use_tc_tiling_on_sc=False),
  )
  def agg_kernel(y_hbm, src_hbm, dst_hbm, zeros_hbm, out_hbm,
                 idx_s, idx_d, buf, acc, sem0, sem1):
    c = lax.axis_index("c")
    s = lax.axis_index("s")
    w = s * NC + c
    pltpu.sync_copy(src_hbm.at[w], idx_s)
    pltpu.sync_copy(dst_hbm.at[w], idx_d)
    sl = pl.ds(s * rows_per_tile, rows_per_tile)
    pltpu.sync_copy(zeros_hbm.at[sl], acc.at[sl])
    pltpu.async_copy(y_hbm.at[idx_s.at[0]], buf.at[0], sem0)
    pltpu.async_copy(y_hbm.at[idx_s.at[1]], buf.at[1], sem1)
    plsc.subcore_barrier()

    def step(k, carry):
      j0 = k * 2
      for b, sem in ((0, sem0), (1, sem1)):
        j = j0 + b
        pltpu.make_async_copy(y_hbm.at[idx_s.at[j]], buf.at[b], sem).wait()
        pltpu.sync_copy(buf.at[b], acc.at[idx_d.at[j]], add=True)
        pltpu.async_copy(y_hbm.at[idx_s.at[j + 2]], buf.at[b], sem)
      return carry

    lax.fori_loop(0, ch // 2, step, 0)
    pltpu.make_async_copy(y_hbm.at[idx_s.at[ch]], buf.at[0], sem0).wait()
    pltpu.make_async_copy(y_hbm.at[idx_s.at[ch + 1]], buf.at[1], sem1).wait()
    plsc.subcore_barrier()
    pltpu.sync_copy(acc.at[sl], out_hbm.at[c, sl])

  return agg_kernel(y_tbl, src3g, dst3, zeros_tbl)


ROWS = 512


def _dinv_block(degp_ref):
  deg = degp_ref[0, :, 0:1] + degp_ref[1, :, 0:1] + 1.0
  return lax.rsqrt(deg)


def _tc1_body(degp_ref, x_ref, w1_ref, y1_ref):
  dinv = _dinv_block(degp_ref)
  xw = jnp.dot(x_ref[...], w1_ref[...], preferred_element_type=_f32)
  y1_ref[...] = dinv * xw


def _tc2_body(s1_ref, y1_ref, degp_ref, b1_ref, w2_ref, y2_ref):
  dinv = _dinv_block(degp_ref)
  agg = (s1_ref[0] + s1_ref[1] + y1_ref[...]) * dinv
  h1 = jnp.maximum(agg + b1_ref[...], 0.0)
  y2_ref[...] = dinv * jnp.dot(h1, w2_ref[...], preferred_element_type=_f32)


def _tc3_body(s2_ref, y2_ref, degp_ref, b2_ref, wih0_ref, bi0_ref, bh0_ref,
              wih1_ref, bi1_ref, bh1_ref, fct_ref, fcb_ref, out_ref):
  dinv = _dinv_block(degp_ref)
  agg = (s2_ref[0] + s2_ref[1] + y2_ref[...]) * dinv
  h2 = jnp.maximum(agg + b2_ref[...], 0.0)

  def lstm_step(xin, wih_ref, bi_ref, bh_ref, hdim):
    gates = (jnp.dot(xin, wih_ref[...], preferred_element_type=_f32)
             + bi_ref[...] + bh_ref[...])
    i = jax.nn.sigmoid(gates[:, 0:hdim])
    g = jnp.tanh(gates[:, 2 * hdim:3 * hdim])
    o = jax.nn.sigmoid(gates[:, 3 * hdim:4 * hdim])
    return o * jnp.tanh(i * g)

  hdim = wih1_ref.shape[0]
  h0 = lstm_step(h2, wih0_ref, bi0_ref, bh0_ref, hdim)
  h1 = lstm_step(h0, wih1_ref, bi1_ref, bh1_ref, hdim)
  out_ref[...] = (jnp.dot(h1, fct_ref[...], preferred_element_type=_f32)
                  + fcb_ref[...])


def _row_grid_call(body, out_dims, *args_and_specs):
  args = [a for a, _ in args_and_specs]
  in_specs = [spec for _, spec in args_and_specs]
  grid = N_PAD // ROWS
  return pl.pallas_call(
      body,
      grid=(grid,),
      in_specs=in_specs,
      out_specs=pl.BlockSpec((ROWS, out_dims), lambda i: (i, 0)),
      out_shape=jax.ShapeDtypeStruct((N_PAD, out_dims), _f32),
  )(*args)


def _whole(a):
  nd = a.ndim
  return pl.BlockSpec(a.shape, lambda i, _nd=nd: (0,) * _nd)


def _rows(a):
  return pl.BlockSpec((ROWS,) + a.shape[1:], lambda i: (i,) + (0,) * (a.ndim - 1))


def _parts(a):
  return pl.BlockSpec((NC, ROWS) + a.shape[2:],
                      lambda i: (0, i) + (0,) * (a.ndim - 2))


def kernel(x, edge_index, W1, b1, W2, b2, Wih0, Whh0, bih0, bhh0,
           Wih1, Whh1, bih1, bhh1, fcW, fcb):
  del Whh0, Whh1
  n = x.shape[0]
  d_gnn = W1.shape[1]
  d_out = fcW.shape[0]

  src = edge_index[0].astype(jnp.int32)
  dst = edge_index[1].astype(jnp.int32)
  e = src.shape[0]

  per_op = NW * CHUNK
  ch = -(-e // per_op)
  ch += ch % 2
  e_pad = ch * per_op
  pad_edges = jnp.full((e_pad - e,), N_PAD - 1, jnp.int32)
  src3 = jnp.concatenate([src, pad_edges]).reshape(NW, ch, CHUNK)
  dst3 = jnp.concatenate([dst, pad_edges]).reshape(NW, ch, CHUNK)
  src3g = jnp.pad(src3, ((0, 0), (0, 2), (0, 0)))

  x_pad = jnp.pad(x, ((0, N_PAD - n), (0, 0)))
  zeros16 = jnp.zeros((N_PAD, 16), _f32)
  zeros_d = jnp.zeros((N_PAD, d_gnn), _f32)
  ones_rows = jnp.ones((CHUNK, 16), _f32)

  degp = _sc_degree(dst3, ones_rows, zeros16)

  y1 = _row_grid_call(_tc1_body, d_gnn,
                      (degp, _parts(degp)), (x_pad, _rows(x_pad)),
                      (W1, _whole(W1)))

  s1 = _sc_aggregate(y1, src3g, dst3, zeros_d)

  b1r = b1.reshape(1, -1)
  y2 = _row_grid_call(_tc2_body, d_gnn,
                      (s1, _parts(s1)), (y1, _rows(y1)),
                      (degp, _parts(degp)), (b1r, _whole(b1r)),
                      (W2, _whole(W2)))

  s2 = _sc_aggregate(y2, src3g, dst3, zeros_d)

  b2r = b2.reshape(1, -1)
  wih0t = Wih0.T
  wih1t = Wih1.T
  bi0 = bih0.reshape(1, -1)
  bh0 = bhh0.reshape(1, -1)
  bi1 = bih1.reshape(1, -1)
  bh1 = bhh1.reshape(1, -1)
  d_head = 8
  fct = jnp.pad(fcW.T, ((0, 0), (0, d_head - d_out)))
  fcbp = jnp.pad(fcb, (0, d_head - d_out)).reshape(1, -1)

  out = _row_grid_call(_tc3_body, d_head,
                       (s2, _parts(s2)), (y2, _rows(y2)),
                       (degp, _parts(degp)), (b2r, _whole(b2r)),
                       (wih0t, _whole(wih0t)), (bi0, _whole(bi0)),
                       (bh0, _whole(bh0)),
                       (wih1t, _whole(wih1t)), (bi1, _whole(bi1)),
                       (bh1, _whole(bh1)),
                       (fct, _whole(fct)), (fcbp, _whole(fcbp)))
  return out[:n, :d_out]


# speedup vs baseline: 16.2361x; 1.1372x over previous
import functools

import jax
import jax.numpy as jnp
from jax import lax
from jax.experimental import pallas as pl
from jax.experimental.pallas import tpu as pltpu
from jax.experimental.pallas import tpu_sc as plsc

NC = 2
NS = 16
NW = NC * NS
CHUNK = 128

N_PAD = 10240

_f32 = jnp.float32


def _sc_degree(dst3, ones_rows, zeros_tbl):
  ch = dst3.shape[1]
  rows_per_tile = N_PAD // NS

  @functools.partial(
      pl.kernel,
      out_type=jax.ShapeDtypeStruct((NC, N_PAD, 16), _f32),
      mesh=plsc.VectorSubcoreMesh(core_axis_name="c", subcore_axis_name="s"),
      scratch_types=[
          pltpu.VMEM((ch, CHUNK), jnp.int32),
          pltpu.VMEM((CHUNK, 16), _f32),
          pltpu.VMEM_SHARED((N_PAD, 16), _f32),
          pltpu.SemaphoreType.DMA,
      ],
      compiler_params=pltpu.CompilerParams(use_tc_tiling_on_sc=False),
  )
  def deg_kernel(dst_hbm, ones_hbm, zeros_hbm, out_hbm, idx_v, ones_v, acc,
                 sem):
    c = lax.axis_index("c")
    s = lax.axis_index("s")
    w = s * NC + c
    pltpu.sync_copy(dst_hbm.at[w], idx_v)
    pltpu.sync_copy(ones_hbm, ones_v)
    sl = pl.ds(s * rows_per_tile, rows_per_tile)
    pltpu.sync_copy(zeros_hbm.at[sl], acc.at[sl])
    plsc.subcore_barrier()

    def step(k, carry):
      j0 = k * 8
      for b in range(8):
        pltpu.async_copy(ones_v, acc.at[idx_v.at[j0 + b]], sem, add=True)
      for b in range(8):
        pltpu.make_async_copy(ones_v, acc.at[idx_v.at[j0 + b]], sem).wait()
      return carry

    lax.fori_loop(0, ch // 8, step, 0)
    plsc.subcore_barrier()
    pltpu.sync_copy(acc.at[sl], out_hbm.at[c, sl])

  return deg_kernel(dst3, ones_rows, zeros_tbl)


def _sc_aggregate(y_tbl, src3, dst3, zeros_tbl):
  d = y_tbl.shape[1]
  ch = dst3.shape[1]
  rows_per_tile = N_PAD // NS

  @functools.partial(
      pl.kernel,
      out_type=jax.ShapeDtypeStruct((NC, N_PAD, d), _f32),
      mesh=plsc.VectorSubcoreMesh(core_axis_name="c", subcore_axis_name="s"),
      scratch_types=[
          pltpu.VMEM((ch, CHUNK), jnp.int32),
          pltpu.VMEM((ch, CHUNK), jnp.int32),
          pltpu.VMEM((4, CHUNK, d), _f32),
          pltpu.VMEM_SHARED((N_PAD, d), _f32),
          [pltpu.SemaphoreType.DMA] * 4,
          [pltpu.SemaphoreType.DMA] * 4,
      ],
      compiler_params=pltpu.CompilerParams(use_tc_tiling_on_sc=False),
  )
  def agg_kernel(y_hbm, src_hbm, dst_hbm, zeros_hbm, out_hbm,
                 idx_s, idx_d, buf, acc, sems_g, sems_s):
    c = lax.axis_index("c")
    s = lax.axis_index("s")
    w = s * NC + c
    pltpu.sync_copy(src_hbm.at[w], idx_s)
    pltpu.sync_copy(dst_hbm.at[w], idx_d)
    sl = pl.ds(s * rows_per_tile, rows_per_tile)
    pltpu.sync_copy(zeros_hbm.at[sl], acc.at[sl])

    def gissue(j, r):
      pltpu.async_copy(y_hbm.at[idx_s.at[j]], buf.at[r], sems_g[r])

    def gwait(j, r):
      pltpu.make_async_copy(y_hbm.at[idx_s.at[j]], buf.at[r],
                            sems_g[r]).wait()

    def sissue(j, r):
      pltpu.async_copy(buf.at[r], acc.at[idx_d.at[j]], sems_s[r], add=True)

    def swait(j, r):
      pltpu.make_async_copy(buf.at[r], acc.at[idx_d.at[j]],
                            sems_s[r]).wait()

    gissue(0, 0)
    gissue(1, 1)
    plsc.subcore_barrier()

    for j in (0, 1):
      gwait(j, j)
      sissue(j, j)
      gissue(j + 2, j + 2)

    def step(k, carry):
      j0 = k * 4 + 2
      for b in range(4):
        j = j0 + b
        r = (2 + b) % 4
        gwait(j, r)
        sissue(j, r)
        swait(j - 2, b)
        gissue(j + 2, b)
      return carry

    lax.fori_loop(0, (ch - 4) // 4, step, 0)

    for j in (ch - 2, ch - 1):
      r = j % 4
      gwait(j, r)
      sissue(j, r)
      swait(j - 2, (j + 2) % 4)
    swait(ch - 2, (ch - 2) % 4)
    swait(ch - 1, (ch - 1) % 4)
    plsc.subcore_barrier()
    pltpu.sync_copy(acc.at[sl], out_hbm.at[c, sl])

  return agg_kernel(y_tbl, src3, dst3, zeros_tbl)


ROWS = 512


def _dinv_block(degp_ref):
  deg = degp_ref[0, :, 0:1] + degp_ref[1, :, 0:1] + 1.0
  return lax.rsqrt(deg)


def _tc1_body(degp_ref, x_ref, w1_ref, y1_ref):
  dinv = _dinv_block(degp_ref)
  xw = jnp.dot(x_ref[...], w1_ref[...], preferred_element_type=_f32)
  y1_ref[...] = dinv * xw


def _tc2_body(s1_ref, y1_ref, degp_ref, b1_ref, w2_ref, y2_ref):
  dinv = _dinv_block(degp_ref)
  agg = (s1_ref[0] + s1_ref[1] + y1_ref[...]) * dinv
  h1 = jnp.maximum(agg + b1_ref[...], 0.0)
  y2_ref[...] = dinv * jnp.dot(h1, w2_ref[...], preferred_element_type=_f32)


def _tc3_body(s2_ref, y2_ref, degp_ref, b2_ref, wih0_ref, bi0_ref, bh0_ref,
              wih1_ref, bi1_ref, bh1_ref, fct_ref, fcb_ref, out_ref):
  dinv = _dinv_block(degp_ref)
  agg = (s2_ref[0] + s2_ref[1] + y2_ref[...]) * dinv
  h2 = jnp.maximum(agg + b2_ref[...], 0.0)

  def lstm_step(xin, wih_ref, bi_ref, bh_ref, hdim):
    gates = (jnp.dot(xin, wih_ref[...], preferred_element_type=_f32)
             + bi_ref[...] + bh_ref[...])
    i = jax.nn.sigmoid(gates[:, 0:hdim])
    g = jnp.tanh(gates[:, 2 * hdim:3 * hdim])
    o = jax.nn.sigmoid(gates[:, 3 * hdim:4 * hdim])
    return o * jnp.tanh(i * g)

  hdim = wih1_ref.shape[0]
  h0 = lstm_step(h2, wih0_ref, bi0_ref, bh0_ref, hdim)
  h1 = lstm_step(h0, wih1_ref, bi1_ref, bh1_ref, hdim)
  out_ref[...] = (jnp.dot(h1, fct_ref[...], preferred_element_type=_f32)
                  + fcb_ref[...])


def _row_grid_call(body, out_dims, *args_and_specs):
  args = [a for a, _ in args_and_specs]
  in_specs = [spec for _, spec in args_and_specs]
  grid = N_PAD // ROWS
  return pl.pallas_call(
      body,
      grid=(grid,),
      in_specs=in_specs,
      out_specs=pl.BlockSpec((ROWS, out_dims), lambda i: (i, 0)),
      out_shape=jax.ShapeDtypeStruct((N_PAD, out_dims), _f32),
  )(*args)


def _whole(a):
  nd = a.ndim
  return pl.BlockSpec(a.shape, lambda i, _nd=nd: (0,) * _nd)


def _rows(a):
  return pl.BlockSpec((ROWS,) + a.shape[1:], lambda i: (i,) + (0,) * (a.ndim - 1))


def _parts(a):
  return pl.BlockSpec((NC, ROWS) + a.shape[2:],
                      lambda i: (0, i) + (0,) * (a.ndim - 2))


def kernel(x, edge_index, W1, b1, W2, b2, Wih0, Whh0, bih0, bhh0,
           Wih1, Whh1, bih1, bhh1, fcW, fcb):
  del Whh0, Whh1
  n = x.shape[0]
  d_gnn = W1.shape[1]
  d_out = fcW.shape[0]

  src = edge_index[0].astype(jnp.int32)
  dst = edge_index[1].astype(jnp.int32)
  e = src.shape[0]

  per_op = NW * CHUNK
  ch = -(-e // per_op)
  ch = -(-ch // 8) * 8
  e_pad = ch * per_op
  pad_edges = jnp.full((e_pad - e,), N_PAD - 1, jnp.int32)
  src3 = jnp.concatenate([src, pad_edges]).reshape(NW, ch, CHUNK)
  dst3 = jnp.concatenate([dst, pad_edges]).reshape(NW, ch, CHUNK)

  x_pad = jnp.pad(x, ((0, N_PAD - n), (0, 0)))
  zeros16 = jnp.zeros((N_PAD, 16), _f32)
  zeros_d = jnp.zeros((N_PAD, d_gnn), _f32)
  ones_rows = jnp.ones((CHUNK, 16), _f32)

  degp = _sc_degree(dst3, ones_rows, zeros16)

  y1 = _row_grid_call(_tc1_body, d_gnn,
                      (degp, _parts(degp)), (x_pad, _rows(x_pad)),
                      (W1, _whole(W1)))

  s1 = _sc_aggregate(y1, src3, dst3, zeros_d)

  b1r = b1.reshape(1, -1)
  y2 = _row_grid_call(_tc2_body, d_gnn,
                      (s1, _parts(s1)), (y1, _rows(y1)),
                      (degp, _parts(degp)), (b1r, _whole(b1r)),
                      (W2, _whole(W2)))

  s2 = _sc_aggregate(y2, src3, dst3, zeros_d)

  b2r = b2.reshape(1, -1)
  wih0t = Wih0.T
  wih1t = Wih1.T
  bi0 = bih0.reshape(1, -1)
  bh0 = bhh0.reshape(1, -1)
  bi1 = bih1.reshape(1, -1)
  bh1 = bhh1.reshape(1, -1)
  d_head = 8
  fct = jnp.pad(fcW.T, ((0, 0), (0, d_head - d_out)))
  fcbp = jnp.pad(fcb, (0, d_head - d_out)).reshape(1, -1)

  out = _row_grid_call(_tc3_body, d_head,
                       (s2, _parts(s2)), (y2, _rows(y2)),
                       (degp, _parts(degp)), (b2r, _whole(b2r)),
                       (wih0t, _whole(wih0t)), (bi0, _whole(bi0)),
                       (bh0, _whole(bh0)),
                       (wih1t, _whole(wih1t)), (bi1, _whole(bi1)),
                       (bh1, _whole(bh1)),
                       (fct, _whole(fct)), (fcbp, _whole(fcbp)))
  return out[:n, :d_out]


# speedup vs baseline: 16.3897x; 1.0095x over previous
import functools

import jax
import jax.numpy as jnp
from jax import lax
from jax.experimental import pallas as pl
from jax.experimental.pallas import tpu as pltpu
from jax.experimental.pallas import tpu_sc as plsc

NC = 2
NS = 16
NW = NC * NS
CHUNK = 128

N_PAD = 10240

_f32 = jnp.float32


def _sc_degree(dst3, ones_rows, zeros_tbl):
  ch = dst3.shape[1]
  rows_per_tile = N_PAD // NS

  @functools.partial(
      pl.kernel,
      out_type=jax.ShapeDtypeStruct((NC, N_PAD, 16), _f32),
      mesh=plsc.VectorSubcoreMesh(core_axis_name="c", subcore_axis_name="s"),
      scratch_types=[
          pltpu.VMEM((ch, CHUNK), jnp.int32),
          pltpu.VMEM((CHUNK, 16), _f32),
          pltpu.VMEM_SHARED((N_PAD, 16), _f32),
          pltpu.SemaphoreType.DMA,
      ],
      compiler_params=pltpu.CompilerParams(use_tc_tiling_on_sc=False),
  )
  def deg_kernel(dst_hbm, ones_hbm, zeros_hbm, out_hbm, idx_v, ones_v, acc,
                 sem):
    c = lax.axis_index("c")
    s = lax.axis_index("s")
    w = s * NC + c
    pltpu.sync_copy(dst_hbm.at[w], idx_v)
    pltpu.sync_copy(ones_hbm, ones_v)
    sl = pl.ds(s * rows_per_tile, rows_per_tile)
    pltpu.sync_copy(zeros_hbm.at[sl], acc.at[sl])
    plsc.subcore_barrier()

    def step(k, carry):
      j0 = k * 8
      for b in range(8):
        pltpu.async_copy(ones_v, acc.at[idx_v.at[j0 + b]], sem, add=True)
      for b in range(8):
        pltpu.make_async_copy(ones_v, acc.at[idx_v.at[j0 + b]], sem).wait()
      return carry

    lax.fori_loop(0, ch // 8, step, 0)
    plsc.subcore_barrier()
    pltpu.sync_copy(acc.at[sl], out_hbm.at[c, sl])

  return deg_kernel(dst3, ones_rows, zeros_tbl)


@functools.cache
def _make_sc_aggregate(ch, d):
  rows_per_tile = N_PAD // NS
  ring = 8
  ahead = ring - 2

  @functools.partial(
      pl.kernel,
      out_type=jax.ShapeDtypeStruct((NC, N_PAD, d), _f32),
      mesh=plsc.VectorSubcoreMesh(core_axis_name="c", subcore_axis_name="s"),
      scratch_types=[
          pltpu.VMEM((ch, CHUNK), jnp.int32),
          pltpu.VMEM((ch, CHUNK), jnp.int32),
          pltpu.VMEM((ring, CHUNK, d), _f32),
          pltpu.VMEM_SHARED((N_PAD, d), _f32),
          [pltpu.SemaphoreType.DMA] * ring,
          [pltpu.SemaphoreType.DMA] * ring,
      ],
      compiler_params=pltpu.CompilerParams(use_tc_tiling_on_sc=False),
  )
  def agg_kernel(y_hbm, src_hbm, dst_hbm, zeros_hbm, out_hbm,
                 idx_s, idx_d, buf, acc, sems_g, sems_s):
    c = lax.axis_index("c")
    s = lax.axis_index("s")
    w = s * NC + c
    pltpu.sync_copy(src_hbm.at[w], idx_s)
    pltpu.sync_copy(dst_hbm.at[w], idx_d)
    sl = pl.ds(s * rows_per_tile, rows_per_tile)
    pltpu.sync_copy(zeros_hbm.at[sl], acc.at[sl])

    def gissue(j, r):
      pltpu.async_copy(y_hbm.at[idx_s.at[j]], buf.at[r], sems_g[r])

    def gwait(j, r):
      pltpu.make_async_copy(y_hbm.at[idx_s.at[j]], buf.at[r],
                            sems_g[r]).wait()

    def sissue(j, r):
      pltpu.async_copy(buf.at[r], acc.at[idx_d.at[j]], sems_s[r], add=True)

    def swait(j, r):
      pltpu.make_async_copy(buf.at[r], acc.at[idx_d.at[j]],
                            sems_s[r]).wait()

    for j in range(ahead):
      gissue(j, j)
    plsc.subcore_barrier()

    for j in (0, 1):
      gwait(j, j)
      sissue(j, j)
      gissue(j + ahead, (j + ahead) % ring)

    def step(k, carry):
      j0 = k * ring + 2
      for b in range(ring):
        j = j0 + b
        r = (2 + b) % ring
        rg = (2 + b + ahead) % ring
        gwait(j, r)
        sissue(j, r)
        swait(j - 2, rg)
        gissue(j + ahead, rg)
      return carry

    lax.fori_loop(0, (ch - ahead - 2) // ring, step, 0)

    for j in range(ch - ahead, ch):
      r = j % ring
      gwait(j, r)
      sissue(j, r)
      swait(j - 2, (j - 2) % ring)
    swait(ch - 2, (ch - 2) % ring)
    swait(ch - 1, (ch - 1) % ring)
    plsc.subcore_barrier()
    pltpu.sync_copy(acc.at[sl], out_hbm.at[c, sl])

  return agg_kernel


ROWS = 512


def _dinv_block(degp_ref):
  deg = degp_ref[0, :, 0:1] + degp_ref[1, :, 0:1] + 1.0
  return lax.rsqrt(deg)


def _tc1_body(degp_ref, x_ref, w1_ref, y1_ref):
  dinv = _dinv_block(degp_ref)
  xw = jnp.dot(x_ref[...], w1_ref[...], preferred_element_type=_f32)
  y1_ref[...] = dinv * xw


def _tc2_body(s1_ref, y1_ref, degp_ref, b1_ref, w2_ref, y2_ref):
  dinv = _dinv_block(degp_ref)
  agg = (s1_ref[0] + s1_ref[1] + y1_ref[...]) * dinv
  h1 = jnp.maximum(agg + b1_ref[...], 0.0)
  y2_ref[...] = dinv * jnp.dot(h1, w2_ref[...], preferred_element_type=_f32)


def _tc3_body(s2_ref, y2_ref, degp_ref, b2_ref, wih0_ref, bi0_ref, bh0_ref,
              wih1_ref, bi1_ref, bh1_ref, fct_ref, fcb_ref, out_ref):
  dinv = _dinv_block(degp_ref)
  agg = (s2_ref[0] + s2_ref[1] + y2_ref[...]) * dinv
  h2 = jnp.maximum(agg + b2_ref[...], 0.0)

  def lstm_step(xin, wih_ref, bi_ref, bh_ref, hdim):
    gates = (jnp.dot(xin, wih_ref[...], preferred_element_type=_f32)
             + bi_ref[...] + bh_ref[...])
    i = jax.nn.sigmoid(gates[:, 0:hdim])
    g = jnp.tanh(gates[:, 2 * hdim:3 * hdim])
    o = jax.nn.sigmoid(gates[:, 3 * hdim:4 * hdim])
    return o * jnp.tanh(i * g)

  hdim = wih1_ref.shape[0]
  h0 = lstm_step(h2, wih0_ref, bi0_ref, bh0_ref, hdim)
  h1 = lstm_step(h0, wih1_ref, bi1_ref, bh1_ref, hdim)
  out_ref[...] = (jnp.dot(h1, fct_ref[...], preferred_element_type=_f32)
                  + fcb_ref[...])


def _row_grid_call(body, out_dims, *args_and_specs):
  args = [a for a, _ in args_and_specs]
  in_specs = [spec for _, spec in args_and_specs]
  grid = N_PAD // ROWS
  return pl.pallas_call(
      body,
      grid=(grid,),
      in_specs=in_specs,
      out_specs=pl.BlockSpec((ROWS, out_dims), lambda i: (i, 0)),
      out_shape=jax.ShapeDtypeStruct((N_PAD, out_dims), _f32),
  )(*args)


def _whole(a):
  nd = a.ndim
  return pl.BlockSpec(a.shape, lambda i, _nd=nd: (0,) * _nd)


def _rows(a):
  return pl.BlockSpec((ROWS,) + a.shape[1:], lambda i: (i,) + (0,) * (a.ndim - 1))


def _parts(a):
  return pl.BlockSpec((NC, ROWS) + a.shape[2:],
                      lambda i: (0, i) + (0,) * (a.ndim - 2))


def kernel(x, edge_index, W1, b1, W2, b2, Wih0, Whh0, bih0, bhh0,
           Wih1, Whh1, bih1, bhh1, fcW, fcb):
  del Whh0, Whh1
  n = x.shape[0]
  d_gnn = W1.shape[1]
  d_out = fcW.shape[0]

  src = edge_index[0].astype(jnp.int32)
  dst = edge_index[1].astype(jnp.int32)
  e = src.shape[0]

  per_op = NW * CHUNK
  ch = -(-e // per_op)
  ch = -(-ch // 8) * 8
  e_pad = ch * per_op
  pad_edges = jnp.full((e_pad - e,), N_PAD - 1, jnp.int32)
  src3 = jnp.concatenate([src, pad_edges]).reshape(NW, ch, CHUNK)
  dst3 = jnp.concatenate([dst, pad_edges]).reshape(NW, ch, CHUNK)

  x_pad = jnp.pad(x, ((0, N_PAD - n), (0, 0)))
  zeros16 = jnp.zeros((N_PAD, 16), _f32)
  zeros_d = jnp.zeros((N_PAD, d_gnn), _f32)
  ones_rows = jnp.ones((CHUNK, 16), _f32)

  degp = _sc_degree(dst3, ones_rows, zeros16)

  y1 = _row_grid_call(_tc1_body, d_gnn,
                      (degp, _parts(degp)), (x_pad, _rows(x_pad)),
                      (W1, _whole(W1)))

  agg_fn = _make_sc_aggregate(ch, d_gnn)
  s1 = agg_fn(y1, src3, dst3, zeros_d)

  b1r = b1.reshape(1, -1)
  y2 = _row_grid_call(_tc2_body, d_gnn,
                      (s1, _parts(s1)), (y1, _rows(y1)),
                      (degp, _parts(degp)), (b1r, _whole(b1r)),
                      (W2, _whole(W2)))

  s2 = agg_fn(y2, src3, dst3, zeros_d)

  b2r = b2.reshape(1, -1)
  wih0t = Wih0.T
  wih1t = Wih1.T
  bi0 = bih0.reshape(1, -1)
  bh0 = bhh0.reshape(1, -1)
  bi1 = bih1.reshape(1, -1)
  bh1 = bhh1.reshape(1, -1)
  d_head = 8
  fct = jnp.pad(fcW.T, ((0, 0), (0, d_head - d_out)))
  fcbp = jnp.pad(fcb, (0, d_head - d_out)).reshape(1, -1)

  out = _row_grid_call(_tc3_body, d_head,
                       (s2, _parts(s2)), (y2, _rows(y2)),
                       (degp, _parts(degp)), (b2r, _whole(b2r)),
                       (wih0t, _whole(wih0t)), (bi0, _whole(bi0)),
                       (bh0, _whole(bh0)),
                       (wih1t, _whole(wih1t)), (bi1, _whole(bi1)),
                       (bh1, _whole(bh1)),
                       (fct, _whole(fct)), (fcbp, _whole(fcbp)))
  return out[:n, :d_out]


# speedup vs baseline: 17.2022x; 1.0496x over previous
import functools

import jax
import jax.numpy as jnp
from jax import lax
from jax.experimental import pallas as pl
from jax.experimental.pallas import tpu as pltpu
from jax.experimental.pallas import tpu_sc as plsc

NC = 2
NS = 16
NW = NC * NS
CHUNK = 128

N_PAD = 10240

_f32 = jnp.float32


def _core_chunks(c, ch0, ch1):
  return jnp.where(c == 0, jnp.int32(ch0), jnp.int32(ch1))


def _sc_degree(dst3, ones_rows, zeros_tbl, ch0, ch1):
  rows_per_tile = N_PAD // NS

  @functools.partial(
      pl.kernel,
      out_type=jax.ShapeDtypeStruct((NC, N_PAD, 16), _f32),
      mesh=plsc.VectorSubcoreMesh(core_axis_name="c", subcore_axis_name="s"),
      scratch_types=[
          pltpu.VMEM((ch0, CHUNK), jnp.int32),
          pltpu.VMEM((CHUNK, 16), _f32),
          pltpu.VMEM_SHARED((N_PAD, 16), _f32),
          pltpu.SemaphoreType.DMA,
      ],
      compiler_params=pltpu.CompilerParams(use_tc_tiling_on_sc=False),
  )
  def deg_kernel(dst_hbm, ones_hbm, zeros_hbm, out_hbm, idx_v, ones_v, acc,
                 sem):
    c = lax.axis_index("c")
    s = lax.axis_index("s")
    w = c * NS + s
    chc = _core_chunks(c, ch0, ch1)
    pltpu.sync_copy(dst_hbm.at[w], idx_v)
    pltpu.sync_copy(ones_hbm, ones_v)
    sl = pl.ds(s * rows_per_tile, rows_per_tile)
    pltpu.sync_copy(zeros_hbm.at[sl], acc.at[sl])
    plsc.subcore_barrier()

    def step(k, carry):
      j0 = k * 8
      for b in range(8):
        pltpu.async_copy(ones_v, acc.at[idx_v.at[j0 + b]], sem, add=True)
      for b in range(8):
        pltpu.make_async_copy(ones_v, acc.at[idx_v.at[j0 + b]], sem).wait()
      return carry

    lax.fori_loop(0, chc // 8, step, 0)
    plsc.subcore_barrier()
    pltpu.sync_copy(acc.at[sl], out_hbm.at[c, sl])

  return deg_kernel(dst3, ones_rows, zeros_tbl)


@functools.cache
def _make_sc_aggregate(ch0, ch1, d):
  rows_per_tile = N_PAD // NS
  ring = 4
  ahead = ring - 2

  @functools.partial(
      pl.kernel,
      out_type=jax.ShapeDtypeStruct((NC, N_PAD, d), _f32),
      mesh=plsc.VectorSubcoreMesh(core_axis_name="c", subcore_axis_name="s"),
      scratch_types=[
          pltpu.VMEM((ch0, CHUNK), jnp.int32),
          pltpu.VMEM((ch0, CHUNK), jnp.int32),
          pltpu.VMEM((ring, CHUNK, d), _f32),
          pltpu.VMEM_SHARED((N_PAD, d), _f32),
          [pltpu.SemaphoreType.DMA] * ring,
          [pltpu.SemaphoreType.DMA] * ring,
      ],
      compiler_params=pltpu.CompilerParams(use_tc_tiling_on_sc=False),
  )
  def agg_kernel(y_hbm, src_hbm, dst_hbm, zeros_hbm, out_hbm,
                 idx_s, idx_d, buf, acc, sems_g, sems_s):
    c = lax.axis_index("c")
    s = lax.axis_index("s")
    w = c * NS + s
    chc = _core_chunks(c, ch0, ch1)
    pltpu.sync_copy(src_hbm.at[w], idx_s)
    pltpu.sync_copy(dst_hbm.at[w], idx_d)
    sl = pl.ds(s * rows_per_tile, rows_per_tile)
    pltpu.sync_copy(zeros_hbm.at[sl], acc.at[sl])

    def gissue(j, r):
      pltpu.async_copy(y_hbm.at[idx_s.at[j]], buf.at[r], sems_g[r])

    def gwait(j, r):
      pltpu.make_async_copy(y_hbm.at[idx_s.at[j]], buf.at[r],
                            sems_g[r]).wait()

    def sissue(j, r):
      pltpu.async_copy(buf.at[r], acc.at[idx_d.at[j]], sems_s[r], add=True)

    def swait(j, r):
      pltpu.make_async_copy(buf.at[r], acc.at[idx_d.at[j]],
                            sems_s[r]).wait()

    for j in range(ahead):
      gissue(j, j)
    plsc.subcore_barrier()

    for j in (0, 1):
      gwait(j, j)
      sissue(j, j)
      gissue(j + ahead, (j + ahead) % ring)

    def step(k, carry):
      j0 = k * ring + 2
      for b in range(ring):
        j = j0 + b
        r = (2 + b) % ring
        rg = (2 + b + ahead) % ring
        gwait(j, r)
        sissue(j, r)
        swait(j - 2, rg)
        gissue(j + ahead, rg)
      return carry

    lax.fori_loop(0, (chc - ahead - 2) // ring, step, 0)

    for i in range(ahead):
      j = chc - ahead + i
      r = (ring - ahead + i) % ring
      gwait(j, r)
      sissue(j, r)
      swait(j - 2, (r - 2) % ring)
    swait(chc - 2, (ring - 2) % ring)
    swait(chc - 1, (ring - 1) % ring)
    plsc.subcore_barrier()
    pltpu.sync_copy(acc.at[sl], out_hbm.at[c, sl])

  return agg_kernel


ROWS = 512


def _dinv_block(degp_ref):
  deg = degp_ref[0, :, 0:1] + degp_ref[1, :, 0:1] + 1.0
  return lax.rsqrt(deg)


def _tc1_body(degp_ref, x_ref, w1_ref, y1_ref):
  dinv = _dinv_block(degp_ref)
  xw = jnp.dot(x_ref[...], w1_ref[...], preferred_element_type=_f32)
  y1_ref[...] = dinv * xw


def _tc2_body(s1_ref, y1_ref, degp_ref, b1_ref, w2_ref, y2_ref):
  dinv = _dinv_block(degp_ref)
  agg = (s1_ref[0] + s1_ref[1] + y1_ref[...]) * dinv
  h1 = jnp.maximum(agg + b1_ref[...], 0.0)
  y2_ref[...] = dinv * jnp.dot(h1, w2_ref[...], preferred_element_type=_f32)


def _tc3_body(s2_ref, y2_ref, degp_ref, b2_ref, wih0_ref, bi0_ref, bh0_ref,
              wih1_ref, bi1_ref, bh1_ref, fct_ref, fcb_ref, out_ref):
  dinv = _dinv_block(degp_ref)
  agg = (s2_ref[0] + s2_ref[1] + y2_ref[...]) * dinv
  h2 = jnp.maximum(agg + b2_ref[...], 0.0)

  def lstm_step(xin, wih_ref, bi_ref, bh_ref, hdim):
    gates = (jnp.dot(xin, wih_ref[...], preferred_element_type=_f32)
             + bi_ref[...] + bh_ref[...])
    i = jax.nn.sigmoid(gates[:, 0:hdim])
    g = jnp.tanh(gates[:, 2 * hdim:3 * hdim])
    o = jax.nn.sigmoid(gates[:, 3 * hdim:4 * hdim])
    return o * jnp.tanh(i * g)

  hdim = wih1_ref.shape[0]
  h0 = lstm_step(h2, wih0_ref, bi0_ref, bh0_ref, hdim)
  h1 = lstm_step(h0, wih1_ref, bi1_ref, bh1_ref, hdim)
  out_ref[...] = (jnp.dot(h1, fct_ref[...], preferred_element_type=_f32)
                  + fcb_ref[...])


def _row_grid_call(body, out_dims, *args_and_specs):
  args = [a for a, _ in args_and_specs]
  in_specs = [spec for _, spec in args_and_specs]
  grid = N_PAD // ROWS
  return pl.pallas_call(
      body,
      grid=(grid,),
      in_specs=in_specs,
      out_specs=pl.BlockSpec((ROWS, out_dims), lambda i: (i, 0)),
      out_shape=jax.ShapeDtypeStruct((N_PAD, out_dims), _f32),
  )(*args)


def _whole(a):
  nd = a.ndim
  return pl.BlockSpec(a.shape, lambda i, _nd=nd: (0,) * _nd)


def _rows(a):
  return pl.BlockSpec((ROWS,) + a.shape[1:], lambda i: (i,) + (0,) * (a.ndim - 1))


def _parts(a):
  return pl.BlockSpec((NC, ROWS) + a.shape[2:],
                      lambda i: (0, i) + (0,) * (a.ndim - 2))


def kernel(x, edge_index, W1, b1, W2, b2, Wih0, Whh0, bih0, bhh0,
           Wih1, Whh1, bih1, bhh1, fcW, fcb):
  del Whh0, Whh1
  n = x.shape[0]
  d_gnn = W1.shape[1]
  d_out = fcW.shape[0]

  src = edge_index[0].astype(jnp.int32)
  dst = edge_index[1].astype(jnp.int32)
  e = src.shape[0]

  split = 0.76
  cht = -(-e // CHUNK)
  c0pt = -(-int(cht * split) // NS)
  ch0 = max(8, -(-c0pt // 8) * 8)
  e0 = min(e, NS * c0pt * CHUNK)
  c1pt = -(-(-(-(e - e0) // CHUNK)) // NS)
  ch1 = max(8, -(-c1pt // 8) * 8)

  n_trash = N_PAD - n
  def pad_to(idx, cap, trash_dst):
    pad_len = cap - idx.shape[0]
    if trash_dst:
      pad = n + (jnp.arange(pad_len, dtype=jnp.int32) % n_trash)
    else:
      pad = jnp.zeros((pad_len,), jnp.int32)
    return jnp.concatenate([idx, pad])

  cap0 = NS * ch0 * CHUNK
  cap1 = NS * ch1 * CHUNK
  src3 = jnp.concatenate([
      pad_to(src[:e0], cap0, False).reshape(NS, ch0, CHUNK),
      jnp.pad(pad_to(src[e0:], cap1, False).reshape(NS, ch1, CHUNK),
              ((0, 0), (0, ch0 - ch1), (0, 0))),
  ])
  dst3 = jnp.concatenate([
      pad_to(dst[:e0], cap0, True).reshape(NS, ch0, CHUNK),
      jnp.pad(pad_to(dst[e0:], cap1, True).reshape(NS, ch1, CHUNK),
              ((0, 0), (0, ch0 - ch1), (0, 0))),
  ])

  x_pad = jnp.pad(x, ((0, N_PAD - n), (0, 0)))
  zeros16 = jnp.zeros((N_PAD, 16), _f32)
  zeros_d = jnp.zeros((N_PAD, d_gnn), _f32)
  ones_rows = jnp.ones((CHUNK, 16), _f32)

  degp = _sc_degree(dst3, ones_rows, zeros16, ch0, ch1)

  y1 = _row_grid_call(_tc1_body, d_gnn,
                      (degp, _parts(degp)), (x_pad, _rows(x_pad)),
                      (W1, _whole(W1)))

  agg_fn = _make_sc_aggregate(ch0, ch1, d_gnn)
  s1 = agg_fn(y1, src3, dst3, zeros_d)

  b1r = b1.reshape(1, -1)
  y2 = _row_grid_call(_tc2_body, d_gnn,
                      (s1, _parts(s1)), (y1, _rows(y1)),
                      (degp, _parts(degp)), (b1r, _whole(b1r)),
                      (W2, _whole(W2)))

  s2 = agg_fn(y2, src3, dst3, zeros_d)

  b2r = b2.reshape(1, -1)
  wih0t = Wih0.T
  wih1t = Wih1.T
  bi0 = bih0.reshape(1, -1)
  bh0 = bhh0.reshape(1, -1)
  bi1 = bih1.reshape(1, -1)
  bh1 = bhh1.reshape(1, -1)
  d_head = 8
  fct = jnp.pad(fcW.T, ((0, 0), (0, d_head - d_out)))
  fcbp = jnp.pad(fcb, (0, d_head - d_out)).reshape(1, -1)

  out = _row_grid_call(_tc3_body, d_head,
                       (s2, _parts(s2)), (y2, _rows(y2)),
                       (degp, _parts(degp)), (b2r, _whole(b2r)),
                       (wih0t, _whole(wih0t)), (bi0, _whole(bi0)),
                       (bh0, _whole(bh0)),
                       (wih1t, _whole(wih1t)), (bi1, _whole(bi1)),
                       (bh1, _whole(bh1)),
                       (fct, _whole(fct)), (fcbp, _whole(fcbp)))
  return out[:n, :d_out]


# speedup vs baseline: 31.6007x; 1.8370x over previous
import functools

import jax
import jax.numpy as jnp
from jax import lax
from jax.experimental import pallas as pl
from jax.experimental.pallas import tpu as pltpu
from jax.experimental.pallas import tpu_sc as plsc

NC = 2
NS = 16
NW = NC * NS
CHUNK = 128

N_PAD = 10240

_f32 = jnp.float32


def _core_chunks(c, ch0, ch1):
  return jnp.where(c == 0, jnp.int32(ch0), jnp.int32(ch1))


def _sc_degree(dst3, ones_rows, zeros_tbl, ch0, ch1):
  rows_per_tile = N_PAD // NS

  @functools.partial(
      pl.kernel,
      out_type=jax.ShapeDtypeStruct((NC, N_PAD, 16), _f32),
      mesh=plsc.VectorSubcoreMesh(core_axis_name="c", subcore_axis_name="s"),
      scratch_types=[
          pltpu.VMEM((ch0, CHUNK), jnp.int32),
          pltpu.VMEM((CHUNK, 16), _f32),
          pltpu.VMEM_SHARED((N_PAD, 16), _f32),
          pltpu.SemaphoreType.DMA,
      ],
      compiler_params=pltpu.CompilerParams(use_tc_tiling_on_sc=False),
  )
  def deg_kernel(dst_hbm, ones_hbm, zeros_hbm, out_hbm, idx_v, ones_v, acc,
                 sem):
    c = lax.axis_index("c")
    s = lax.axis_index("s")
    w = c * NS + s
    chc = _core_chunks(c, ch0, ch1)
    pltpu.sync_copy(dst_hbm.at[w], idx_v)
    pltpu.sync_copy(ones_hbm, ones_v)
    sl = pl.ds(s * rows_per_tile, rows_per_tile)
    pltpu.sync_copy(zeros_hbm.at[sl], acc.at[sl])
    plsc.subcore_barrier()

    def step(k, carry):
      j0 = k * 8
      for b in range(8):
        pltpu.async_copy(ones_v, acc.at[idx_v.at[j0 + b]], sem, add=True)
      for b in range(8):
        pltpu.make_async_copy(ones_v, acc.at[idx_v.at[j0 + b]], sem).wait()
      return carry

    lax.fori_loop(0, chc // 8, step, 0)
    plsc.subcore_barrier()
    pltpu.sync_copy(acc.at[sl], out_hbm.at[c, sl])

  return deg_kernel(dst3, ones_rows, zeros_tbl)


@functools.cache
def _make_sc_aggregate(ch0, ch1, d):
  del ch1
  ch = ch0
  rows_per_tile = N_PAD // NS
  ring = 2

  @functools.partial(
      pl.kernel,
      out_type=jax.ShapeDtypeStruct((NC, N_PAD, d), _f32),
      mesh=plsc.VectorSubcoreMesh(core_axis_name="c", subcore_axis_name="s"),
      scratch_types=[
          pltpu.VMEM((ch, CHUNK), jnp.int32),
          pltpu.VMEM((ch, CHUNK), jnp.int32),
          pltpu.VMEM((ring, CHUNK, d), _f32),
          pltpu.VMEM_SHARED((N_PAD, d), _f32),
          pltpu.VMEM_SHARED((N_PAD, d), _f32),
          [pltpu.SemaphoreType.DMA] * ring,
          [pltpu.SemaphoreType.DMA] * ring,
      ],
      compiler_params=pltpu.CompilerParams(use_tc_tiling_on_sc=False),
  )
  def agg_kernel(y_hbm, src_hbm, dst_hbm, zeros_hbm, out_hbm,
                 idx_s, idx_d, buf, acc, ytbl, sems_g, sems_s):
    c = lax.axis_index("c")
    s = lax.axis_index("s")
    w = c * NS + s
    pltpu.sync_copy(src_hbm.at[w], idx_s)
    pltpu.sync_copy(dst_hbm.at[w], idx_d)
    sl = pl.ds(s * rows_per_tile, rows_per_tile)
    pltpu.sync_copy(y_hbm.at[sl], ytbl.at[sl])
    pltpu.sync_copy(zeros_hbm.at[sl], acc.at[sl])

    def gissue(j, r):
      pltpu.async_copy(ytbl.at[idx_s.at[j]], buf.at[r], sems_g[r])

    def gwait(j, r):
      pltpu.make_async_copy(ytbl.at[idx_s.at[j]], buf.at[r],
                            sems_g[r]).wait()

    def sissue(j, r):
      pltpu.async_copy(buf.at[r], acc.at[idx_d.at[j]], sems_s[r], add=True)

    def swait(j, r):
      pltpu.make_async_copy(buf.at[r], acc.at[idx_d.at[j]],
                            sems_s[r]).wait()

    plsc.subcore_barrier()

    gissue(0, 0)
    gwait(0, 0)
    sissue(0, 0)
    gissue(1, 1)

    def step(k, carry):
      j0 = k * 2 + 1
      for bp in range(2):
        j = j0 + bp
        r = (1 + bp) % 2
        gwait(j, r)
        sissue(j, r)
        swait(j - 1, 1 - r)
        gissue(j + 1, 1 - r)
      return carry

    lax.fori_loop(0, (ch - 2) // 2, step, 0)

    gwait(ch - 1, 1)
    sissue(ch - 1, 1)
    swait(ch - 2, 0)
    swait(ch - 1, 1)
    plsc.subcore_barrier()
    pltpu.sync_copy(acc.at[sl], out_hbm.at[c, sl])

  return agg_kernel


ROWS = 512


def _dinv_block(degp_ref):
  deg = degp_ref[0, :, 0:1] + degp_ref[1, :, 0:1] + 1.0
  return lax.rsqrt(deg)


def _tc1_body(degp_ref, x_ref, w1_ref, y1_ref):
  dinv = _dinv_block(degp_ref)
  xw = jnp.dot(x_ref[...], w1_ref[...], preferred_element_type=_f32)
  y1_ref[...] = dinv * xw


def _tc2_body(s1_ref, y1_ref, degp_ref, b1_ref, w2_ref, y2_ref):
  dinv = _dinv_block(degp_ref)
  agg = (s1_ref[0] + s1_ref[1] + y1_ref[...]) * dinv
  h1 = jnp.maximum(agg + b1_ref[...], 0.0)
  y2_ref[...] = dinv * jnp.dot(h1, w2_ref[...], preferred_element_type=_f32)


def _tc3_body(s2_ref, y2_ref, degp_ref, b2_ref, wih0_ref, bi0_ref, bh0_ref,
              wih1_ref, bi1_ref, bh1_ref, fct_ref, fcb_ref, out_ref):
  dinv = _dinv_block(degp_ref)
  agg = (s2_ref[0] + s2_ref[1] + y2_ref[...]) * dinv
  h2 = jnp.maximum(agg + b2_ref[...], 0.0)

  def lstm_step(xin, wih_ref, bi_ref, bh_ref, hdim):
    gates = (jnp.dot(xin, wih_ref[...], preferred_element_type=_f32)
             + bi_ref[...] + bh_ref[...])
    i = jax.nn.sigmoid(gates[:, 0:hdim])
    g = jnp.tanh(gates[:, 2 * hdim:3 * hdim])
    o = jax.nn.sigmoid(gates[:, 3 * hdim:4 * hdim])
    return o * jnp.tanh(i * g)

  hdim = wih1_ref.shape[0]
  h0 = lstm_step(h2, wih0_ref, bi0_ref, bh0_ref, hdim)
  h1 = lstm_step(h0, wih1_ref, bi1_ref, bh1_ref, hdim)
  out_ref[...] = (jnp.dot(h1, fct_ref[...], preferred_element_type=_f32)
                  + fcb_ref[...])


def _row_grid_call(body, out_dims, *args_and_specs):
  args = [a for a, _ in args_and_specs]
  in_specs = [spec for _, spec in args_and_specs]
  grid = N_PAD // ROWS
  return pl.pallas_call(
      body,
      grid=(grid,),
      in_specs=in_specs,
      out_specs=pl.BlockSpec((ROWS, out_dims), lambda i: (i, 0)),
      out_shape=jax.ShapeDtypeStruct((N_PAD, out_dims), _f32),
  )(*args)


def _whole(a):
  nd = a.ndim
  return pl.BlockSpec(a.shape, lambda i, _nd=nd: (0,) * _nd)


def _rows(a):
  return pl.BlockSpec((ROWS,) + a.shape[1:], lambda i: (i,) + (0,) * (a.ndim - 1))


def _parts(a):
  return pl.BlockSpec((NC, ROWS) + a.shape[2:],
                      lambda i: (0, i) + (0,) * (a.ndim - 2))


def kernel(x, edge_index, W1, b1, W2, b2, Wih0, Whh0, bih0, bhh0,
           Wih1, Whh1, bih1, bhh1, fcW, fcb):
  del Whh0, Whh1
  n = x.shape[0]
  d_gnn = W1.shape[1]
  d_out = fcW.shape[0]

  src = edge_index[0].astype(jnp.int32)
  dst = edge_index[1].astype(jnp.int32)
  e = src.shape[0]

  per_op = NW * CHUNK
  ch = -(-e // per_op)
  ch = -(-ch // 8) * 8
  e_pad = ch * per_op
  n_trash = N_PAD - n
  pad_src = jnp.zeros((e_pad - e,), jnp.int32)
  pad_dst = n + (jnp.arange(e_pad - e, dtype=jnp.int32) % n_trash)
  src3 = jnp.concatenate([src, pad_src]).reshape(NW, ch, CHUNK)
  dst3 = jnp.concatenate([dst, pad_dst]).reshape(NW, ch, CHUNK)

  x_pad = jnp.pad(x, ((0, N_PAD - n), (0, 0)))
  zeros16 = jnp.zeros((N_PAD, 16), _f32)
  zeros_d = jnp.zeros((N_PAD, d_gnn), _f32)
  ones_rows = jnp.ones((CHUNK, 16), _f32)

  degp = _sc_degree(dst3, ones_rows, zeros16, ch, ch)

  y1 = _row_grid_call(_tc1_body, d_gnn,
                      (degp, _parts(degp)), (x_pad, _rows(x_pad)),
                      (W1, _whole(W1)))

  agg_fn = _make_sc_aggregate(ch, ch, d_gnn)
  s1 = agg_fn(y1, src3, dst3, zeros_d)

  b1r = b1.reshape(1, -1)
  y2 = _row_grid_call(_tc2_body, d_gnn,
                      (s1, _parts(s1)), (y1, _rows(y1)),
                      (degp, _parts(degp)), (b1r, _whole(b1r)),
                      (W2, _whole(W2)))

  s2 = agg_fn(y2, src3, dst3, zeros_d)

  b2r = b2.reshape(1, -1)
  wih0t = Wih0.T
  wih1t = Wih1.T
  bi0 = bih0.reshape(1, -1)
  bh0 = bhh0.reshape(1, -1)
  bi1 = bih1.reshape(1, -1)
  bh1 = bhh1.reshape(1, -1)
  d_head = 8
  fct = jnp.pad(fcW.T, ((0, 0), (0, d_head - d_out)))
  fcbp = jnp.pad(fcb, (0, d_head - d_out)).reshape(1, -1)

  out = _row_grid_call(_tc3_body, d_head,
                       (s2, _parts(s2)), (y2, _rows(y2)),
                       (degp, _parts(degp)), (b2r, _whole(b2r)),
                       (wih0t, _whole(wih0t)), (bi0, _whole(bi0)),
                       (bh0, _whole(bh0)),
                       (wih1t, _whole(wih1t)), (bi1, _whole(bi1)),
                       (bh1, _whole(bh1)),
                       (fct, _whole(fct)), (fcbp, _whole(fcbp)))
  return out[:n, :d_out]


# speedup vs baseline: 31.7622x; 1.0051x over previous
import functools

import jax
import jax.numpy as jnp
from jax import lax
from jax.experimental import pallas as pl
from jax.experimental.pallas import tpu as pltpu
from jax.experimental.pallas import tpu_sc as plsc

NC = 2
NS = 16
NW = NC * NS
CHUNK = 128

N_PAD = 10240

_f32 = jnp.float32


def _core_chunks(c, ch0, ch1):
  return jnp.where(c == 0, jnp.int32(ch0), jnp.int32(ch1))


def _sc_degree(dst3, ones_rows, zeros_tbl, ch0, ch1):
  rows_per_tile = N_PAD // NS

  @functools.partial(
      pl.kernel,
      out_type=jax.ShapeDtypeStruct((NC, N_PAD, 16), _f32),
      mesh=plsc.VectorSubcoreMesh(core_axis_name="c", subcore_axis_name="s"),
      scratch_types=[
          pltpu.VMEM((ch0, CHUNK), jnp.int32),
          pltpu.VMEM((CHUNK, 16), _f32),
          pltpu.VMEM_SHARED((N_PAD, 16), _f32),
          pltpu.SemaphoreType.DMA,
      ],
      compiler_params=pltpu.CompilerParams(use_tc_tiling_on_sc=False),
  )
  def deg_kernel(dst_hbm, ones_hbm, zeros_hbm, out_hbm, idx_v, ones_v, acc,
                 sem):
    c = lax.axis_index("c")
    s = lax.axis_index("s")
    w = c * NS + s
    chc = _core_chunks(c, ch0, ch1)
    pltpu.sync_copy(dst_hbm.at[w], idx_v)
    pltpu.sync_copy(ones_hbm, ones_v)
    sl = pl.ds(s * rows_per_tile, rows_per_tile)
    pltpu.sync_copy(zeros_hbm.at[sl], acc.at[sl])
    plsc.subcore_barrier()

    def step(k, carry):
      j0 = k * 8
      for b in range(8):
        pltpu.async_copy(ones_v, acc.at[idx_v.at[j0 + b]], sem, add=True)
      for b in range(8):
        pltpu.make_async_copy(ones_v, acc.at[idx_v.at[j0 + b]], sem).wait()
      return carry

    lax.fori_loop(0, chc // 8, step, 0)
    plsc.subcore_barrier()
    pltpu.sync_copy(acc.at[sl], out_hbm.at[c, sl])

  return deg_kernel(dst3, ones_rows, zeros_tbl)


@functools.cache
def _make_sc_aggregate(ch0, ch1, d):
  del ch1
  ch = ch0
  rows_per_tile = N_PAD // NS
  ring = 2

  @functools.partial(
      pl.kernel,
      out_type=jax.ShapeDtypeStruct((NC, N_PAD, d), _f32),
      mesh=plsc.VectorSubcoreMesh(core_axis_name="c", subcore_axis_name="s"),
      scratch_types=[
          pltpu.VMEM((ch, CHUNK), jnp.int32),
          pltpu.VMEM((ch, CHUNK), jnp.int32),
          pltpu.VMEM((ring, CHUNK, d), _f32),
          pltpu.VMEM_SHARED((N_PAD, d), _f32),
          pltpu.VMEM_SHARED((N_PAD, d), _f32),
          [pltpu.SemaphoreType.DMA] * ring,
          [pltpu.SemaphoreType.DMA] * ring,
      ],
      compiler_params=pltpu.CompilerParams(use_tc_tiling_on_sc=False),
  )
  def agg_kernel(y_hbm, src_hbm, dst_hbm, zeros_hbm, out_hbm,
                 idx_s, idx_d, buf, acc, ytbl, sems_g, sems_s):
    c = lax.axis_index("c")
    s = lax.axis_index("s")
    w = c * NS + s
    pltpu.sync_copy(src_hbm.at[w], idx_s)
    pltpu.sync_copy(dst_hbm.at[w], idx_d)
    sl = pl.ds(s * rows_per_tile, rows_per_tile)
    pltpu.sync_copy(y_hbm.at[sl], ytbl.at[sl])
    pltpu.sync_copy(zeros_hbm.at[sl], acc.at[sl])

    def gissue(j, r):
      pltpu.async_copy(ytbl.at[idx_s.at[j]], buf.at[r], sems_g[r])

    def gwait(j, r):
      pltpu.make_async_copy(ytbl.at[idx_s.at[j]], buf.at[r],
                            sems_g[r]).wait()

    def sissue(j, r):
      pltpu.async_copy(buf.at[r], acc.at[idx_d.at[j]], sems_s[r], add=True)

    def swait(j, r):
      pltpu.make_async_copy(buf.at[r], acc.at[idx_d.at[j]],
                            sems_s[r]).wait()

    plsc.subcore_barrier()

    gissue(0, 0)
    gwait(0, 0)
    sissue(0, 0)
    gissue(1, 1)

    def step(k, carry):
      j0 = k * 2 + 1
      for bp in range(2):
        j = j0 + bp
        r = (1 + bp) % 2
        gwait(j, r)
        sissue(j, r)
        swait(j - 1, 1 - r)
        gissue(j + 1, 1 - r)
      return carry

    lax.fori_loop(0, (ch - 2) // 2, step, 0)

    gwait(ch - 1, 1)
    sissue(ch - 1, 1)
    swait(ch - 2, 0)
    swait(ch - 1, 1)
    plsc.subcore_barrier()
    pltpu.sync_copy(acc.at[sl], out_hbm.at[c, sl])

  return agg_kernel


ROWS = 512


def _dinv_block(degp_ref):
  deg = degp_ref[0, :, 0:1] + degp_ref[1, :, 0:1] + 1.0
  return lax.rsqrt(deg)


def _tc1_body(degp_ref, x_ref, w1_ref, y1_ref, dinv_ref):
  dinv = _dinv_block(degp_ref)
  xw = jnp.dot(x_ref[...], w1_ref[...], preferred_element_type=_f32)
  y1_ref[...] = dinv * xw
  dinv_ref[...] = jnp.broadcast_to(dinv, y1_ref.shape)


def _tc2_body(s1_ref, y1_ref, dinv_ref, b1_ref, w2_ref, y2_ref):
  dinv = dinv_ref[...]
  agg = (s1_ref[0] + s1_ref[1] + y1_ref[...]) * dinv
  h1 = jnp.maximum(agg + b1_ref[...], 0.0)
  y2_ref[...] = dinv * jnp.dot(h1, w2_ref[...], preferred_element_type=_f32)


def _tc3_body(s2_ref, y2_ref, dinv_ref, b2_ref, wih0_ref, bi0_ref, bh0_ref,
              wih1_ref, bi1_ref, bh1_ref, fct_ref, fcb_ref, out_ref):
  dinv = dinv_ref[...]
  agg = (s2_ref[0] + s2_ref[1] + y2_ref[...]) * dinv
  h2 = jnp.maximum(agg + b2_ref[...], 0.0)

  def lstm_step(xin, wih_ref, bi_ref, bh_ref, hdim):
    gates = (jnp.dot(xin, wih_ref[...], preferred_element_type=_f32)
             + bi_ref[...] + bh_ref[...])
    i = jax.nn.sigmoid(gates[:, 0:hdim])
    g = jnp.tanh(gates[:, 2 * hdim:3 * hdim])
    o = jax.nn.sigmoid(gates[:, 3 * hdim:4 * hdim])
    return o * jnp.tanh(i * g)

  hdim = wih1_ref.shape[0]
  h0 = lstm_step(h2, wih0_ref, bi0_ref, bh0_ref, hdim)
  h1 = lstm_step(h0, wih1_ref, bi1_ref, bh1_ref, hdim)
  out_ref[...] = (jnp.dot(h1, fct_ref[...], preferred_element_type=_f32)
                  + fcb_ref[...])


def _row_grid_call(body, out_dims, *args_and_specs):
  args = [a for a, _ in args_and_specs]
  in_specs = [spec for _, spec in args_and_specs]
  grid = N_PAD // ROWS
  return pl.pallas_call(
      body,
      grid=(grid,),
      in_specs=in_specs,
      out_specs=pl.BlockSpec((ROWS, out_dims), lambda i: (i, 0)),
      out_shape=jax.ShapeDtypeStruct((N_PAD, out_dims), _f32),
  )(*args)


def _whole(a):
  nd = a.ndim
  return pl.BlockSpec(a.shape, lambda i, _nd=nd: (0,) * _nd)


def _rows(a):
  return pl.BlockSpec((ROWS,) + a.shape[1:], lambda i: (i,) + (0,) * (a.ndim - 1))


def _parts(a):
  return pl.BlockSpec((NC, ROWS) + a.shape[2:],
                      lambda i: (0, i) + (0,) * (a.ndim - 2))


def kernel(x, edge_index, W1, b1, W2, b2, Wih0, Whh0, bih0, bhh0,
           Wih1, Whh1, bih1, bhh1, fcW, fcb):
  del Whh0, Whh1
  n = x.shape[0]
  d_gnn = W1.shape[1]
  d_out = fcW.shape[0]

  src = edge_index[0].astype(jnp.int32)
  dst = edge_index[1].astype(jnp.int32)
  e = src.shape[0]

  per_op = NW * CHUNK
  ch = -(-e // per_op)
  ch = -(-ch // 8) * 8
  e_pad = ch * per_op
  n_trash = N_PAD - n
  pad_src = jnp.zeros((e_pad - e,), jnp.int32)
  pad_dst = n + (jnp.arange(e_pad - e, dtype=jnp.int32) % n_trash)
  src3 = jnp.concatenate([src, pad_src]).reshape(NW, ch, CHUNK)
  dst3 = jnp.concatenate([dst, pad_dst]).reshape(NW, ch, CHUNK)

  x_pad = jnp.pad(x, ((0, N_PAD - n), (0, 0)))
  zeros16 = jnp.zeros((N_PAD, 16), _f32)
  zeros_d = jnp.zeros((N_PAD, d_gnn), _f32)
  ones_rows = jnp.ones((CHUNK, 16), _f32)

  degp = _sc_degree(dst3, ones_rows, zeros16, ch, ch)

  y1, dinv64 = pl.pallas_call(
      _tc1_body,
      grid=(N_PAD // ROWS,),
      in_specs=[_parts(degp), _rows(x_pad), _whole(W1)],
      out_specs=[pl.BlockSpec((ROWS, d_gnn), lambda i: (i, 0))] * 2,
      out_shape=[jax.ShapeDtypeStruct((N_PAD, d_gnn), _f32)] * 2,
  )(degp, x_pad, W1)

  agg_fn = _make_sc_aggregate(ch, ch, d_gnn)
  s1 = agg_fn(y1, src3, dst3, zeros_d)

  b1r = b1.reshape(1, -1)
  y2 = _row_grid_call(_tc2_body, d_gnn,
                      (s1, _parts(s1)), (y1, _rows(y1)),
                      (dinv64, _rows(dinv64)), (b1r, _whole(b1r)),
                      (W2, _whole(W2)))

  s2 = agg_fn(y2, src3, dst3, zeros_d)

  b2r = b2.reshape(1, -1)
  wih0t = Wih0.T
  wih1t = Wih1.T
  bi0 = bih0.reshape(1, -1)
  bh0 = bhh0.reshape(1, -1)
  bi1 = bih1.reshape(1, -1)
  bh1 = bhh1.reshape(1, -1)
  d_head = 8
  fct = jnp.pad(fcW.T, ((0, 0), (0, d_head - d_out)))
  fcbp = jnp.pad(fcb, (0, d_head - d_out)).reshape(1, -1)

  out = _row_grid_call(_tc3_body, d_head,
                       (s2, _parts(s2)), (y2, _rows(y2)),
                       (dinv64, _rows(dinv64)), (b2r, _whole(b2r)),
                       (wih0t, _whole(wih0t)), (bi0, _whole(bi0)),
                       (bh0, _whole(bh0)),
                       (wih1t, _whole(wih1t)), (bi1, _whole(bi1)),
                       (bh1, _whole(bh1)),
                       (fct, _whole(fct)), (fcbp, _whole(fcbp)))
  return out[:n, :d_out]


# speedup vs baseline: 32.0925x; 1.0104x over previous
import functools

import jax
import jax.numpy as jnp
from jax import lax
from jax.experimental import pallas as pl
from jax.experimental.pallas import tpu as pltpu
from jax.experimental.pallas import tpu_sc as plsc

NC = 2
NS = 16
NW = NC * NS
CHUNK = 128

N_PAD = 10240

_f32 = jnp.float32


def _core_chunks(c, ch0, ch1):
  return jnp.where(c == 0, jnp.int32(ch0), jnp.int32(ch1))


def _sc_degree(dst3, ones_rows, zeros_tbl, ch0, ch1):
  rows_per_tile = N_PAD // NS

  @functools.partial(
      pl.kernel,
      out_type=jax.ShapeDtypeStruct((NC, N_PAD, 16), _f32),
      mesh=plsc.VectorSubcoreMesh(core_axis_name="c", subcore_axis_name="s"),
      scratch_types=[
          pltpu.VMEM((ch0, CHUNK), jnp.int32),
          pltpu.VMEM((CHUNK, 16), _f32),
          pltpu.VMEM_SHARED((N_PAD, 16), _f32),
          pltpu.SemaphoreType.DMA,
      ],
      compiler_params=pltpu.CompilerParams(use_tc_tiling_on_sc=False),
  )
  def deg_kernel(dst_hbm, ones_hbm, zeros_hbm, out_hbm, idx_v, ones_v, acc,
                 sem):
    c = lax.axis_index("c")
    s = lax.axis_index("s")
    w = c * NS + s
    chc = _core_chunks(c, ch0, ch1)
    pltpu.sync_copy(dst_hbm.at[w], idx_v)
    pltpu.sync_copy(ones_hbm, ones_v)
    sl = pl.ds(s * rows_per_tile, rows_per_tile)
    pltpu.sync_copy(zeros_hbm.at[sl], acc.at[sl])
    plsc.subcore_barrier()

    def step(k, carry):
      j0 = k * 8
      for b in range(8):
        pltpu.async_copy(ones_v, acc.at[idx_v.at[j0 + b]], sem, add=True)
      for b in range(8):
        pltpu.make_async_copy(ones_v, acc.at[idx_v.at[j0 + b]], sem).wait()
      return carry

    lax.fori_loop(0, chc // 8, step, 0)
    plsc.subcore_barrier()
    pltpu.sync_copy(acc.at[sl], out_hbm.at[c, sl])

  return deg_kernel(dst3, ones_rows, zeros_tbl)


@functools.cache
def _make_sc_aggregate(ch0, ch1, d):
  del ch1
  ch = ch0
  rows_per_tile = N_PAD // NS
  ring = 2

  @functools.partial(
      pl.kernel,
      out_type=jax.ShapeDtypeStruct((NC, N_PAD, d), _f32),
      mesh=plsc.VectorSubcoreMesh(core_axis_name="c", subcore_axis_name="s"),
      scratch_types=[
          pltpu.VMEM((ch, CHUNK), jnp.int32),
          pltpu.VMEM((ch, CHUNK), jnp.int32),
          pltpu.VMEM((ring, CHUNK, d), _f32),
          pltpu.VMEM_SHARED((N_PAD, d), _f32),
          pltpu.VMEM_SHARED((N_PAD, d), _f32),
          [pltpu.SemaphoreType.DMA] * ring,
          [pltpu.SemaphoreType.DMA] * ring,
      ],
      compiler_params=pltpu.CompilerParams(use_tc_tiling_on_sc=False),
  )
  def agg_kernel(y_hbm, src_hbm, dst_hbm, out_hbm,
                 idx_s, idx_d, buf, acc, ytbl, sems_g, sems_s):
    c = lax.axis_index("c")
    s = lax.axis_index("s")
    w = c * NS + s
    pltpu.sync_copy(src_hbm.at[w], idx_s)
    pltpu.sync_copy(dst_hbm.at[w], idx_d)
    sl = pl.ds(s * rows_per_tile, rows_per_tile)
    pltpu.sync_copy(y_hbm.at[sl], ytbl.at[sl])

    def zstep(i, carry):
      for k in range(d // 16):
        buf[0, i, pl.ds(k * 16, 16)] = jnp.zeros((16,), _f32)
      return carry

    lax.fori_loop(0, CHUNK, zstep, 0)
    for t in range(rows_per_tile // CHUNK):
      pltpu.sync_copy(buf.at[0],
                      acc.at[pl.ds(s * rows_per_tile + t * CHUNK, CHUNK)])

    def gissue(j, r):
      pltpu.async_copy(ytbl.at[idx_s.at[j]], buf.at[r], sems_g[r])

    def gwait(j, r):
      pltpu.make_async_copy(ytbl.at[idx_s.at[j]], buf.at[r],
                            sems_g[r]).wait()

    def sissue(j, r):
      pltpu.async_copy(buf.at[r], acc.at[idx_d.at[j]], sems_s[r], add=True)

    def swait(j, r):
      pltpu.make_async_copy(buf.at[r], acc.at[idx_d.at[j]],
                            sems_s[r]).wait()

    plsc.subcore_barrier()

    gissue(0, 0)
    gwait(0, 0)
    sissue(0, 0)
    gissue(1, 1)

    def step(k, carry):
      j0 = k * 2 + 1
      for bp in range(2):
        j = j0 + bp
        r = (1 + bp) % 2
        gwait(j, r)
        sissue(j, r)
        swait(j - 1, 1 - r)
        gissue(j + 1, 1 - r)
      return carry

    lax.fori_loop(0, (ch - 2) // 2, step, 0)

    gwait(ch - 1, 1)
    sissue(ch - 1, 1)
    swait(ch - 2, 0)
    swait(ch - 1, 1)
    plsc.subcore_barrier()
    pltpu.sync_copy(acc.at[sl], out_hbm.at[c, sl])

  return agg_kernel


ROWS = 512


def _dinv_block(degp_ref):
  deg = degp_ref[0, :, 0:1] + degp_ref[1, :, 0:1] + 1.0
  return lax.rsqrt(deg)


def _tc1_body(degp_ref, x_ref, w1_ref, y1_ref, dinv_ref):
  dinv = _dinv_block(degp_ref)
  xw = jnp.dot(x_ref[...], w1_ref[...], preferred_element_type=_f32)
  y1_ref[...] = dinv * xw
  dinv_ref[...] = jnp.broadcast_to(dinv, y1_ref.shape)


def _tc2_body(s1_ref, y1_ref, dinv_ref, b1_ref, w2_ref, y2_ref):
  dinv = dinv_ref[...]
  agg = (s1_ref[0] + s1_ref[1] + y1_ref[...]) * dinv
  h1 = jnp.maximum(agg + b1_ref[...], 0.0)
  y2_ref[...] = dinv * jnp.dot(h1, w2_ref[...], preferred_element_type=_f32)


def _tc3_body(s2_ref, y2_ref, dinv_ref, b2_ref, wih0_ref, bi0_ref, bh0_ref,
              wih1_ref, bi1_ref, bh1_ref, fct_ref, fcb_ref, out_ref):
  dinv = dinv_ref[...]
  agg = (s2_ref[0] + s2_ref[1] + y2_ref[...]) * dinv
  h2 = jnp.maximum(agg + b2_ref[...], 0.0)

  def lstm_step(xin, wih_ref, bi_ref, bh_ref, hdim):
    gates = (jnp.dot(xin, wih_ref[...], preferred_element_type=_f32)
             + bi_ref[...] + bh_ref[...])
    i = jax.nn.sigmoid(gates[:, 0:hdim])
    g = jnp.tanh(gates[:, 2 * hdim:3 * hdim])
    o = jax.nn.sigmoid(gates[:, 3 * hdim:4 * hdim])
    return o * jnp.tanh(i * g)

  hdim = wih1_ref.shape[0]
  h0 = lstm_step(h2, wih0_ref, bi0_ref, bh0_ref, hdim)
  h1 = lstm_step(h0, wih1_ref, bi1_ref, bh1_ref, hdim)
  out_ref[...] = (jnp.dot(h1, fct_ref[...], preferred_element_type=_f32)
                  + fcb_ref[...])


def _row_grid_call(body, out_dims, *args_and_specs):
  args = [a for a, _ in args_and_specs]
  in_specs = [spec for _, spec in args_and_specs]
  grid = N_PAD // ROWS
  return pl.pallas_call(
      body,
      grid=(grid,),
      in_specs=in_specs,
      out_specs=pl.BlockSpec((ROWS, out_dims), lambda i: (i, 0)),
      out_shape=jax.ShapeDtypeStruct((N_PAD, out_dims), _f32),
  )(*args)


def _whole(a):
  nd = a.ndim
  return pl.BlockSpec(a.shape, lambda i, _nd=nd: (0,) * _nd)


def _rows(a):
  return pl.BlockSpec((ROWS,) + a.shape[1:], lambda i: (i,) + (0,) * (a.ndim - 1))


def _parts(a):
  return pl.BlockSpec((NC, ROWS) + a.shape[2:],
                      lambda i: (0, i) + (0,) * (a.ndim - 2))


def kernel(x, edge_index, W1, b1, W2, b2, Wih0, Whh0, bih0, bhh0,
           Wih1, Whh1, bih1, bhh1, fcW, fcb):
  del Whh0, Whh1
  n = x.shape[0]
  d_gnn = W1.shape[1]
  d_out = fcW.shape[0]

  src = edge_index[0].astype(jnp.int32)
  dst = edge_index[1].astype(jnp.int32)
  e = src.shape[0]

  per_op = NW * CHUNK
  ch = -(-e // per_op)
  ch = -(-ch // 8) * 8
  e_pad = ch * per_op
  n_trash = N_PAD - n
  pad_src = jnp.zeros((e_pad - e,), jnp.int32)
  pad_dst = n + (jnp.arange(e_pad - e, dtype=jnp.int32) % n_trash)
  src3 = jnp.concatenate([src, pad_src]).reshape(NW, ch, CHUNK)
  dst3 = jnp.concatenate([dst, pad_dst]).reshape(NW, ch, CHUNK)

  x_pad = jnp.pad(x, ((0, N_PAD - n), (0, 0)))
  zeros16 = jnp.zeros((N_PAD, 16), _f32)
  ones_rows = jnp.ones((CHUNK, 16), _f32)

  degp = _sc_degree(dst3, ones_rows, zeros16, ch, ch)

  y1, dinv64 = pl.pallas_call(
      _tc1_body,
      grid=(N_PAD // ROWS,),
      in_specs=[_parts(degp), _rows(x_pad), _whole(W1)],
      out_specs=[pl.BlockSpec((ROWS, d_gnn), lambda i: (i, 0))] * 2,
      out_shape=[jax.ShapeDtypeStruct((N_PAD, d_gnn), _f32)] * 2,
  )(degp, x_pad, W1)

  agg_fn = _make_sc_aggregate(ch, ch, d_gnn)
  s1 = agg_fn(y1, src3, dst3)

  b1r = b1.reshape(1, -1)
  y2 = _row_grid_call(_tc2_body, d_gnn,
                      (s1, _parts(s1)), (y1, _rows(y1)),
                      (dinv64, _rows(dinv64)), (b1r, _whole(b1r)),
                      (W2, _whole(W2)))

  s2 = agg_fn(y2, src3, dst3)

  b2r = b2.reshape(1, -1)
  wih0t = Wih0.T
  wih1t = Wih1.T
  bi0 = bih0.reshape(1, -1)
  bh0 = bhh0.reshape(1, -1)
  bi1 = bih1.reshape(1, -1)
  bh1 = bhh1.reshape(1, -1)
  d_head = 8
  fct = jnp.pad(fcW.T, ((0, 0), (0, d_head - d_out)))
  fcbp = jnp.pad(fcb, (0, d_head - d_out)).reshape(1, -1)

  out = _row_grid_call(_tc3_body, d_head,
                       (s2, _parts(s2)), (y2, _rows(y2)),
                       (dinv64, _rows(dinv64)), (b2r, _whole(b2r)),
                       (wih0t, _whole(wih0t)), (bi0, _whole(bi0)),
                       (bh0, _whole(bh0)),
                       (wih1t, _whole(wih1t)), (bi1, _whole(bi1)),
                       (bh1, _whole(bh1)),
                       (fct, _whole(fct)), (fcbp, _whole(fcbp)))
  return out[:n, :d_out]
